# baseline (speedup 1.0000x reference)
import functools

import jax
import jax.numpy as jnp
from jax.experimental import pallas as pl
from jax.experimental.pallas import tpu as pltpu


def _sigmoid(x):
    return 0.5 * jnp.tanh(0.5 * x) + 0.5


def _fused_kernel(xn_ref, xr_ref, xw_ref, WrT_ref, brb_ref, WihT_ref,
                  WhhT_ref, bg_ref, WcT_ref, bc_ref, gamma_ref, beta_ref,
                  out_ref, *, T, H, DV):
    br = jax.nn.relu(
        jnp.dot(xr_ref[0], WrT_ref[...],
                preferred_element_type=jnp.float32) + brb_ref[...])

    x = xw_ref[0]
    B = x.shape[0]
    lens = jnp.broadcast_to(xn_ref[0][:, 3:4], (B, H))

    xt = jnp.transpose(x, (1, 0, 2)).reshape(T * B, DV)
    xg = jnp.dot(xt, WihT_ref[...],
                 preferred_element_type=jnp.float32) + bg_ref[...]

    h = jnp.zeros((B, H), dtype=jnp.float32)
    c = jnp.zeros((B, H), dtype=jnp.float32)
    WhhT = WhhT_ref[...]

    for t in range(T):
        gates = xg[t * B:(t + 1) * B, :] + jnp.dot(
            h, WhhT, preferred_element_type=jnp.float32)
        i_g = gates[:, 0 * H:1 * H]
        f_g = gates[:, 1 * H:2 * H]
        g_g = gates[:, 2 * H:3 * H]
        o_g = gates[:, 3 * H:4 * H]
        c_new = _sigmoid(f_g) * c + _sigmoid(i_g) * jnp.tanh(g_g)
        h_new = _sigmoid(o_g) * jnp.tanh(c_new)
        m = t < lens
        h = jnp.where(m, h_new, h)
        c = jnp.where(m, c_new, c)

    hb = jnp.concatenate([br, h], axis=1)
    out = jnp.dot(hb, WcT_ref[...],
                  preferred_element_type=jnp.float32) + bc_ref[...]
    mu = jnp.mean(out, axis=1, keepdims=True)
    d = out - mu
    var = jnp.mean(d * d, axis=1, keepdims=True)
    y = d * jax.lax.rsqrt(var + 1e-5) * gamma_ref[...] + beta_ref[...]
    out_ref[0] = y


def kernel(xr, xw, xn, Wr, br_b, W_ih, W_hh, b_ih, b_hh, Wc, bc, gamma, beta):
    BS, SL, DR = xr.shape
    T, DV = xw.shape[2], xw.shape[3]
    H = Wr.shape[0]
    DH = Wc.shape[0]
    B = 256
    nsl = SL // B

    xn32 = xn.astype(jnp.int32)

    WrT = Wr.T
    WihT = W_ih.T
    WhhT = W_hh.T
    bg = (b_ih + b_hh).reshape(1, 4 * H)
    WcT = Wc.T

    out = pl.pallas_call(
        functools.partial(_fused_kernel, T=T, H=H, DV=DV),
        grid=(BS, nsl),
        in_specs=[
            pl.BlockSpec((1, B, 4), lambda b, j: (b, j, 0)),
            pl.BlockSpec((1, B, DR), lambda b, j: (b, j, 0)),
            pl.BlockSpec((1, B, T, DV), lambda b, j: (b, j, 0, 0)),
            pl.BlockSpec((DR, H), lambda b, j: (0, 0)),
            pl.BlockSpec((1, H), lambda b, j: (0, 0)),
            pl.BlockSpec((DV, 4 * H), lambda b, j: (0, 0)),
            pl.BlockSpec((H, 4 * H), lambda b, j: (0, 0)),
            pl.BlockSpec((1, 4 * H), lambda b, j: (0, 0)),
            pl.BlockSpec((DH, DH), lambda b, j: (0, 0)),
            pl.BlockSpec((1, DH), lambda b, j: (0, 0)),
            pl.BlockSpec((1, DH), lambda b, j: (0, 0)),
            pl.BlockSpec((1, DH), lambda b, j: (0, 0)),
        ],
        out_specs=pl.BlockSpec((1, B, DH), lambda b, j: (b, j, 0)),
        out_shape=jax.ShapeDtypeStruct((BS, SL, DH), jnp.float32),
        compiler_params=pltpu.CompilerParams(
            dimension_semantics=("parallel", "parallel")),
    )(xn32, xr, xw, WrT, br_b.reshape(1, H), WihT, WhhT, bg, WcT,
      bc.reshape(1, DH), gamma.reshape(1, DH), beta.reshape(1, DH))

    return out


# speedup vs baseline: 1.0652x; 1.0652x over previous
import functools

import jax
import jax.numpy as jnp
from jax.experimental import pallas as pl
from jax.experimental.pallas import tpu as pltpu


def _sigmoid(x):
    return 0.5 * jnp.tanh(0.5 * x) + 0.5


def _fused_kernel(len_ref, xr_ref, xw_ref, WrT_ref, brb_ref, WihT_ref,
                  WhhT_ref, bg_ref, WcT_ref, bc_ref, gamma_ref, beta_ref,
                  out_ref, *, T, H):
    br = jax.nn.relu(
        jnp.dot(xr_ref[...], WrT_ref[...],
                preferred_element_type=jnp.float32) + brb_ref[...])

    lens = len_ref[...]
    x = xw_ref[...]
    B = x.shape[0]

    xt = jnp.transpose(x, (1, 0, 2)).reshape(T * B, -1)
    xg = jnp.dot(xt, WihT_ref[...],
                 preferred_element_type=jnp.float32) + bg_ref[...]

    h = jnp.zeros((B, H), dtype=jnp.float32)
    c = jnp.zeros((B, H), dtype=jnp.float32)
    WhhT = WhhT_ref[...]

    for t in range(T):
        gates = xg[t * B:(t + 1) * B, :] + jnp.dot(
            h.astype(jnp.bfloat16), WhhT, preferred_element_type=jnp.float32)
        i_g = gates[:, 0 * H:1 * H]
        f_g = gates[:, 1 * H:2 * H]
        g_g = gates[:, 2 * H:3 * H]
        o_g = gates[:, 3 * H:4 * H]
        c_new = _sigmoid(f_g) * c + _sigmoid(i_g) * jnp.tanh(g_g)
        h_new = _sigmoid(o_g) * jnp.tanh(c_new)
        m = t < lens
        h = jnp.where(m, h_new, h)
        c = jnp.where(m, c_new, c)

    hb = jnp.concatenate([br, h], axis=1).astype(jnp.bfloat16)
    out = jnp.dot(hb, WcT_ref[...],
                  preferred_element_type=jnp.float32) + bc_ref[...]
    mu = jnp.mean(out, axis=1, keepdims=True)
    d = out - mu
    var = jnp.mean(d * d, axis=1, keepdims=True)
    y = d * jax.lax.rsqrt(var + 1e-5) * gamma_ref[...] + beta_ref[...]
    out_ref[...] = y


def kernel(xr, xw, xn, Wr, br_b, W_ih, W_hh, b_ih, b_hh, Wc, bc, gamma, beta):
    BS, SL, DR = xr.shape
    T, DV = xw.shape[2], xw.shape[3]
    H = Wr.shape[0]
    DH = Wc.shape[0]
    N = BS * SL
    B = 256
    nblocks = N // B

    xr2 = xr.reshape(N, DR)
    xw2 = xw.reshape(N, T, DV).astype(jnp.bfloat16)
    lens2 = jnp.broadcast_to(
        xn[:, :, -1].reshape(N, 1).astype(jnp.int32), (N, H))

    WrT = Wr.T
    WihT = W_ih.T.astype(jnp.bfloat16)
    WhhT = W_hh.T.astype(jnp.bfloat16)
    bg = (b_ih + b_hh).reshape(1, 4 * H)
    WcT = Wc.T.astype(jnp.bfloat16)

    out = pl.pallas_call(
        functools.partial(_fused_kernel, T=T, H=H),
        grid=(nblocks,),
        in_specs=[
            pl.BlockSpec((B, H), lambda i: (i, 0)),
            pl.BlockSpec((B, DR), lambda i: (i, 0)),
            pl.BlockSpec((B, T, DV), lambda i: (i, 0, 0)),
            pl.BlockSpec((DR, H), lambda i: (0, 0)),
            pl.BlockSpec((1, H), lambda i: (0, 0)),
            pl.BlockSpec((DV, 4 * H), lambda i: (0, 0)),
            pl.BlockSpec((H, 4 * H), lambda i: (0, 0)),
            pl.BlockSpec((1, 4 * H), lambda i: (0, 0)),
            pl.BlockSpec((DH, DH), lambda i: (0, 0)),
            pl.BlockSpec((1, DH), lambda i: (0, 0)),
            pl.BlockSpec((1, DH), lambda i: (0, 0)),
            pl.BlockSpec((1, DH), lambda i: (0, 0)),
        ],
        out_specs=pl.BlockSpec((B, DH), lambda i: (i, 0)),
        out_shape=jax.ShapeDtypeStruct((N, DH), jnp.float32),
        compiler_params=pltpu.CompilerParams(
            dimension_semantics=("parallel",)),
    )(lens2, xr2, xw2, WrT, br_b.reshape(1, H), WihT, WhhT, bg, WcT,
      bc.reshape(1, DH), gamma.reshape(1, DH), beta.reshape(1, DH))

    return out.reshape(BS, SL, DH)
